# Initial kernel scaffold; baseline (speedup 1.0000x reference)
import jax
import jax.numpy as jnp
from jax.experimental import pallas as pl


def kernel(user_embedding, item_embedding, edge_index):
    raise NotImplementedError("write your pallas kernel here")


# speedup vs baseline: 3.2883x; 3.2883x over previous
import functools

import jax
import jax.numpy as jnp
from jax import lax
from jax.experimental import pallas as pl
from jax.experimental.pallas import tpu as pltpu
from jax.experimental.pallas import tpu_sc as plsc

USER_SIZE = 5000
ITEM_SIZE = 5000
N_NODES = USER_SIZE + ITEM_SIZE
H = 256
HH = 128
E = 160000
EROWS = E // 128
NC = 2
NS = 16
NP = 10240
PR = NP // NS
BLK = 80

_MESH = plsc.VectorSubcoreMesh(
    core_axis_name="c", subcore_axis_name="s", num_cores=NC, num_subcores=NS)

_F32 = jnp.float32
_ZV = None


def _edge_bounds(s):
    base = EROWS // NS
    rem = EROWS % NS
    start = s * base + jnp.minimum(s, rem)
    count = base + jnp.where(s < rem, 1, 0)
    return start, count


def _rsqrt16(v):
    i = lax.bitcast_convert_type(v, jnp.int32)
    i = 0x5F3759DF - (i >> 1)
    y = lax.bitcast_convert_type(i, _F32)
    for _ in range(3):
        y = y * (1.5 - 0.5 * v * y * y)
    return y


def _fill2d(buf, rows, val):
    vec = jnp.full((16,), val, _F32)

    def body(r, carry):
        for j in range(8):
            buf[r, pl.ds(j * 16, 16)] = vec
        return carry

    lax.fori_loop(0, rows, body, 0)


def _sget(buf, i):
    return buf[pl.ds(i, 16)][0]


def _scale_rows(src_buf, dst_buf, scale_buf, base, rows):
    def body(r, carry):
        sc = _sget(scale_buf, base + r)
        for j in range(8):
            dst_buf[r, pl.ds(j * 16, 16)] = src_buf[r, pl.ds(j * 16, 16)] * sc
        return carry

    lax.fori_loop(0, rows, body, 0)


def _norms_kernel(src2d, dst2d, xl, xr, norm_out, norm_in, h0l, h0r,
                  dego_s, degi_s, ones_v, sbuf, dbuf, degbuf, nobuf, nibuf,
                  xbuf, zbuf):
    c = lax.axis_index("c")
    s = lax.axis_index("s")
    off = s * PR

    def zb(r, carry):
        zbuf[pl.ds(r * 16, 16)] = jnp.zeros((16,), _F32)
        return carry
    lax.fori_loop(0, PR // 16, zb, 0)
    pltpu.sync_copy(zbuf.at[pl.ds(0, PR)], dego_s.at[pl.ds(off, PR)])
    pltpu.sync_copy(zbuf.at[pl.ds(0, PR)], degi_s.at[pl.ds(off, PR)])
    for j in range(8):
        ones_v[pl.ds(j * 16, 16)] = jnp.ones((16,), _F32)
    plsc.subcore_barrier()

    start, count = _edge_bounds(s)

    def cbody(r, carry):
        pltpu.sync_copy(src2d.at[pl.ds(r, 1)], sbuf)
        pltpu.sync_copy(dst2d.at[pl.ds(r, 1)], dbuf)
        pltpu.sync_copy(ones_v, dego_s.at[sbuf.at[0]], add=True)
        pltpu.sync_copy(ones_v, degi_s.at[dbuf.at[0]], add=True)
        return carry

    lax.fori_loop(start, start + count, cbody, 0)
    plsc.subcore_barrier()

    pltpu.sync_copy(dego_s.at[pl.ds(off, PR)], degbuf)

    def nb_out(j, carry):
        v = jnp.maximum(degbuf[pl.ds(j * 16, 16)], 1.0)
        nobuf[pl.ds(j * 16, 16)] = _rsqrt16(v)
        return carry
    lax.fori_loop(0, PR // 16, nb_out, 0)

    pltpu.sync_copy(degi_s.at[pl.ds(off, PR)], degbuf)

    def nb_in(j, carry):
        v = jnp.maximum(degbuf[pl.ds(j * 16, 16)], 1.0)
        nibuf[pl.ds(j * 16, 16)] = _rsqrt16(v)
        return carry
    lax.fori_loop(0, PR // 16, nb_in, 0)

    @pl.when(c == 0)
    def _():
        pltpu.sync_copy(nobuf.at[pl.ds(0, PR)], norm_out.at[pl.ds(off, PR)])
        pltpu.sync_copy(nibuf.at[pl.ds(0, PR)], norm_in.at[pl.ds(off, PR)])

    def hblk(k, carry):
        r0 = off + k * BLK

        @pl.when(c == 0)
        def _():
            pltpu.sync_copy(xl.at[pl.ds(r0, BLK)], xbuf)
        @pl.when(c == 1)
        def _():
            pltpu.sync_copy(xr.at[pl.ds(r0, BLK)], xbuf)
        _scale_rows(xbuf, xbuf, nobuf, k * BLK, BLK)
        @pl.when(c == 0)
        def _():
            pltpu.sync_copy(xbuf, h0l.at[pl.ds(r0, BLK)])
        @pl.when(c == 1)
        def _():
            pltpu.sync_copy(xbuf, h0r.at[pl.ds(r0, BLK)])
        return carry

    lax.fori_loop(0, PR // BLK, hblk, 0)


def _aggregate(c, s, hl, hr, src2d, dst2d, acc_s, sbuf, dbuf, rowbuf, zbuf):
    _fill2d(zbuf, BLK, 0.0)

    def zb(k, carry):
        pltpu.sync_copy(zbuf, acc_s.at[pl.ds(s * PR + k * BLK, BLK)])
        return carry
    lax.fori_loop(0, PR // BLK, zb, 0)
    plsc.subcore_barrier()

    start, count = _edge_bounds(s)

    def ebody(r, carry):
        pltpu.sync_copy(src2d.at[pl.ds(r, 1)], sbuf)
        pltpu.sync_copy(dst2d.at[pl.ds(r, 1)], dbuf)

        @pl.when(c == 0)
        def _():
            pltpu.sync_copy(hl.at[sbuf.at[0]], rowbuf)
        @pl.when(c == 1)
        def _():
            pltpu.sync_copy(hr.at[sbuf.at[0]], rowbuf)
        pltpu.sync_copy(rowbuf, acc_s.at[dbuf.at[0]], add=True)
        return carry

    lax.fori_loop(start, start + count, ebody, 0)
    plsc.subcore_barrier()


def _layer1_kernel(hl, hr, src2d, dst2d, norm_in, norm_out,
                   e1l, e1r, h1l, h1r,
                   acc_s, sbuf, dbuf, rowbuf, zbuf, nibuf, nobuf):
    c = lax.axis_index("c")
    s = lax.axis_index("s")
    _aggregate(c, s, hl, hr, src2d, dst2d, acc_s, sbuf, dbuf, rowbuf, zbuf)

    off = s * PR
    pltpu.sync_copy(norm_in.at[pl.ds(off, PR)], nibuf.at[pl.ds(0, PR)])
    pltpu.sync_copy(norm_out.at[pl.ds(off, PR)], nobuf.at[pl.ds(0, PR)])

    def fblk(k, carry):
        r0 = off + k * BLK
        pltpu.sync_copy(acc_s.at[pl.ds(r0, BLK)], zbuf)
        _scale_rows(zbuf, zbuf, nibuf, k * BLK, BLK)

        @pl.when(c == 0)
        def _():
            pltpu.sync_copy(zbuf, e1l.at[pl.ds(r0, BLK)])
        @pl.when(c == 1)
        def _():
            pltpu.sync_copy(zbuf, e1r.at[pl.ds(r0, BLK)])
        _scale_rows(zbuf, zbuf, nobuf, k * BLK, BLK)

        @pl.when(c == 0)
        def _():
            pltpu.sync_copy(zbuf, h1l.at[pl.ds(r0, BLK)])
        @pl.when(c == 1)
        def _():
            pltpu.sync_copy(zbuf, h1r.at[pl.ds(r0, BLK)])
        return carry

    lax.fori_loop(0, PR // BLK, fblk, 0)


def _layer2_kernel(hl, hr, src2d, dst2d, norm_in, x0l, x0r, e1l, e1r,
                   outl, outr,
                   acc_s, sbuf, dbuf, rowbuf, zbuf, e1buf, xbuf, nibuf):
    c = lax.axis_index("c")
    s = lax.axis_index("s")
    _aggregate(c, s, hl, hr, src2d, dst2d, acc_s, sbuf, dbuf, rowbuf, zbuf)

    off = s * PR
    pltpu.sync_copy(norm_in.at[pl.ds(off, PR)], nibuf.at[pl.ds(0, PR)])

    def fblk(k, carry):
        r0 = off + k * BLK
        pltpu.sync_copy(acc_s.at[pl.ds(r0, BLK)], zbuf)

        @pl.when(c == 0)
        def _():
            pltpu.sync_copy(x0l.at[pl.ds(r0, BLK)], xbuf)
            pltpu.sync_copy(e1l.at[pl.ds(r0, BLK)], e1buf)
        @pl.when(c == 1)
        def _():
            pltpu.sync_copy(x0r.at[pl.ds(r0, BLK)], xbuf)
            pltpu.sync_copy(e1r.at[pl.ds(r0, BLK)], e1buf)

        def rbody(r, carry):
            ni = _sget(nibuf, k * BLK + r)
            for j in range(8):
                sl = pl.ds(j * 16, 16)
                xbuf[r, sl] = (xbuf[r, sl] + 0.5 * e1buf[r, sl]
                               + (ni * (1.0 / 3.0)) * zbuf[r, sl])
            return carry
        lax.fori_loop(0, BLK, rbody, 0)

        @pl.when(c == 0)
        def _():
            pltpu.sync_copy(xbuf, outl.at[pl.ds(r0, BLK)])
        @pl.when(c == 1)
        def _():
            pltpu.sync_copy(xbuf, outr.at[pl.ds(r0, BLK)])
        return carry

    lax.fori_loop(0, PR // BLK, fblk, 0)


def _sds(shape, dtype=_F32):
    return jax.ShapeDtypeStruct(shape, dtype)


_norms_call = pl.kernel(
    _norms_kernel,
    out_type=(_sds((NP,)), _sds((NP,)), _sds((NP, HH)), _sds((NP, HH))),
    mesh=_MESH,
    scratch_types=[
        pltpu.VMEM_SHARED((NP,), _F32),
        pltpu.VMEM_SHARED((NP,), _F32),
        pltpu.VMEM((128,), _F32),
        pltpu.VMEM((1, 128), jnp.int32),
        pltpu.VMEM((1, 128), jnp.int32),
        pltpu.VMEM((PR,), _F32),
        pltpu.VMEM((PR + 16,), _F32),
        pltpu.VMEM((PR + 16,), _F32),
        pltpu.VMEM((BLK, HH), _F32),
        pltpu.VMEM((PR,), _F32),
    ],
    name="gcn_norms_h0",
)

_layer1_call = pl.kernel(
    _layer1_kernel,
    out_type=(_sds((NP, HH)), _sds((NP, HH)), _sds((NP, HH)), _sds((NP, HH))),
    mesh=_MESH,
    scratch_types=[
        pltpu.VMEM_SHARED((NP, HH), _F32),
        pltpu.VMEM((1, 128), jnp.int32),
        pltpu.VMEM((1, 128), jnp.int32),
        pltpu.VMEM((128, HH), _F32),
        pltpu.VMEM((BLK, HH), _F32),
        pltpu.VMEM((PR + 16,), _F32),
        pltpu.VMEM((PR + 16,), _F32),
    ],
    name="gcn_layer1",
)

_layer2_call = pl.kernel(
    _layer2_kernel,
    out_type=(_sds((NP, HH)), _sds((NP, HH))),
    mesh=_MESH,
    scratch_types=[
        pltpu.VMEM_SHARED((NP, HH), _F32),
        pltpu.VMEM((1, 128), jnp.int32),
        pltpu.VMEM((1, 128), jnp.int32),
        pltpu.VMEM((128, HH), _F32),
        pltpu.VMEM((BLK, HH), _F32),
        pltpu.VMEM((BLK, HH), _F32),
        pltpu.VMEM((BLK, HH), _F32),
        pltpu.VMEM((PR + 16,), _F32),
    ],
    name="gcn_layer2_final",
)


def kernel(user_embedding, item_embedding, edge_index):
    x0 = jnp.concatenate([user_embedding, item_embedding], axis=0)
    x0p = jnp.zeros((NP, H), _F32).at[:N_NODES].set(x0)
    x0l = x0p[:, :HH]
    x0r = x0p[:, HH:]
    src2d = edge_index[0].astype(jnp.int32).reshape(EROWS, 128)
    dst2d = edge_index[1].astype(jnp.int32).reshape(EROWS, 128)

    norm_out, norm_in, h0l, h0r = _norms_call(src2d, dst2d, x0l, x0r)
    e1l, e1r, h1l, h1r = _layer1_call(h0l, h0r, src2d, dst2d, norm_in, norm_out)
    outl, outr = _layer2_call(h1l, h1r, src2d, dst2d, norm_in, x0l, x0r,
                              e1l, e1r)

    full = jnp.concatenate([outl[:N_NODES], outr[:N_NODES]], axis=1)
    return full[:USER_SIZE], full[USER_SIZE:]
